# baseline (speedup 1.0000x reference)
import jax
import jax.numpy as jnp
from jax.experimental import pallas as pl

N = 10000
E = 320000
DF = 128
DE = 16
H = 64
OUT = 128


def _elu(v):
    return jnp.maximum(v, 0.0) + jnp.exp(jnp.minimum(v, 0.0)) - 1.0


def _front_xa_body(x_ref, w_ref, b_ref, o_ref):
    o_ref[...] = jnp.dot(x_ref[...], w_ref[...],
                         preferred_element_type=jnp.float32) + b_ref[...]


def _edge_body(g_ref, ea_ref, w_ref, o_ref):
    t = _elu(g_ref[...]
             + jnp.dot(ea_ref[...], w_ref[...],
                       preferred_element_type=jnp.float32))
    o_ref[...] = jnp.concatenate(
        [t, jnp.ones((t.shape[0], 1), jnp.float32)], axis=1)


def _tail_body(s_ref, x_ref, w1b_ref, b1b_ref, w2a_ref, b2a_ref,
               w2b_ref, b2b_ref, o_ref):
    cnt = s_ref[:, H:H + 1]
    mean_elu = s_ref[:, 0:H] / jnp.maximum(cnt, 1.0)
    gate = jnp.where(cnt > 0.5, 1.0, 0.0)
    m1 = jnp.dot(mean_elu, w1b_ref[...],
                 preferred_element_type=jnp.float32) + gate * b1b_ref[...]
    hx = (jnp.dot(x_ref[...], w2a_ref[0:DF, :],
                  preferred_element_type=jnp.float32)
          + jnp.dot(m1, w2a_ref[DF:DF + H, :],
                    preferred_element_type=jnp.float32)
          + b2a_ref[...])
    h = _elu(hx)
    o_ref[...] = jnp.dot(h, w2b_ref[...],
                         preferred_element_type=jnp.float32) + b2b_ref[...]


def kernel(x, edge_index, edge_attr, u, batch,
           W1a, b1a, W1b, b1b, W2a, b2a, W2b, b2b):
    row = edge_index[0]
    col = edge_index[1]

    xa = pl.pallas_call(
        _front_xa_body,
        out_shape=jax.ShapeDtypeStruct((N, H), jnp.float32),
    )(x, W1a[:DF], b1a.reshape(1, H))

    from jax.experimental.compute_on import compute_on

    @jax.jit
    @compute_on("tpu_sparsecore")
    def _sc_gather(tbl, idx):
        return jnp.take(tbl, idx, axis=0, mode="clip")

    g = _sc_gather(xa, row)

    EB = 4000
    t = pl.pallas_call(
        _edge_body,
        grid=(E // EB,),
        in_specs=[pl.BlockSpec((EB, H), lambda i: (i, 0)),
                  pl.BlockSpec((EB, DE), lambda i: (i, 0)),
                  pl.BlockSpec((DE, H), lambda i: (0, 0))],
        out_specs=pl.BlockSpec((EB, H + 1), lambda i: (i, 0)),
        out_shape=jax.ShapeDtypeStruct((E, H + 1), jnp.float32),
    )(g, edge_attr, W1a[DF:])

    @jax.jit
    @compute_on("tpu_sparsecore")
    def _sc_segsum(vals, idx):
        return jax.ops.segment_sum(vals, idx, num_segments=N)

    s = _sc_segsum(t, col)

    NB = 1000
    out = pl.pallas_call(
        _tail_body,
        grid=(N // NB,),
        in_specs=[pl.BlockSpec((NB, H + 1), lambda i: (i, 0)),
                  pl.BlockSpec((NB, DF), lambda i: (i, 0)),
                  pl.BlockSpec((H, H), lambda i: (0, 0)),
                  pl.BlockSpec((1, H), lambda i: (0, 0)),
                  pl.BlockSpec((DF + H, H), lambda i: (0, 0)),
                  pl.BlockSpec((1, H), lambda i: (0, 0)),
                  pl.BlockSpec((H, OUT), lambda i: (0, 0)),
                  pl.BlockSpec((1, OUT), lambda i: (0, 0))],
        out_specs=pl.BlockSpec((NB, OUT), lambda i: (i, 0)),
        out_shape=jax.ShapeDtypeStruct((N, OUT), jnp.float32),
    )(s, x, W1b, b1b.reshape(1, H), W2a, b2a.reshape(1, H),
      W2b, b2b.reshape(1, OUT))

    return out
